# baseline (speedup 1.0000x reference)
import functools

import jax
import jax.numpy as jnp
from jax import lax
from jax.experimental import pallas as pl
from jax.experimental.pallas import tpu as pltpu
from jax.experimental.pallas import tpu_sc as plsc

N = 10000
E = 320000
CIN = 128
COUT = 128
NC = 2
NS = 16
EPW = E // (NC * NS)
EPT = E // NS
CH = 80
NPT = N // NS

f32 = jnp.float32
i32 = jnp.int32

_mesh = plsc.VectorSubcoreMesh(
    core_axis_name="c", subcore_axis_name="s", num_cores=NC, num_subcores=NS)


_BLK = 2000


def _prep_body(x_ref, we_ref, be_ref, wl_ref, p1_ref, t2_ref):
  xb = x_ref[...]
  dn = (((1,), (1,)), ((), ()))
  w1 = we_ref[:, :CIN]
  w2 = we_ref[:, CIN:]
  p1_ref[...] = lax.dot_general(xb, w1, dn, preferred_element_type=f32)
  t2_ref[:, :COUT] = (
      lax.dot_general(xb, w2, dn, preferred_element_type=f32) + be_ref[...])
  t2_ref[:, COUT:] = lax.dot_general(
      xb, wl_ref[...], dn, preferred_element_type=f32)


def _prep(x, W_edge, be2d, W_lin):
  return pl.pallas_call(
      _prep_body,
      grid=(N // _BLK,),
      in_specs=[
          pl.BlockSpec((_BLK, CIN), lambda i: (i, 0)),
          pl.BlockSpec((COUT, 2 * CIN), lambda i: (0, 0)),
          pl.BlockSpec((1, COUT), lambda i: (0, 0)),
          pl.BlockSpec((COUT, CIN), lambda i: (0, 0)),
      ],
      out_specs=[
          pl.BlockSpec((_BLK, COUT), lambda i: (i, 0)),
          pl.BlockSpec((_BLK, 2 * COUT), lambda i: (i, 0)),
      ],
      out_shape=[
          jax.ShapeDtypeStruct((N, COUT), f32),
          jax.ShapeDtypeStruct((N, 2 * COUT), f32),
      ],
  )(x, W_edge, be2d, W_lin)


def _k2_body(esrc, edst, T_out, deg_out, sbuf, dbuf, kbuf, ebuf, onesb, dsl,
             deg_sh, sem):
  c = lax.axis_index("c")
  s = lax.axis_index("s")
  w = c * NS + s

  @pl.loop(0, 40, unroll=True)
  def _(i):
    dsl[pl.ds(i * 16, 16)] = jnp.zeros((16,), f32)
  pltpu.sync_copy(dsl, deg_sh.at[pl.ds(s * 624, 640)])

  @pl.loop(0, CH // 16, unroll=True)
  def _(j):
    onesb[pl.ds(j * 16, 16)] = jnp.ones((16,), f32)

  plsc.subcore_barrier()

  base0 = w * EPW

  @pl.loop(0, EPW // CH)
  def _(g):
    b = base0 + g * CH
    pltpu.sync_copy(esrc.at[pl.ds(b, CH)], sbuf)
    pltpu.sync_copy(edst.at[pl.ds(b, CH)], dbuf)

    @pl.loop(0, CH // 16, unroll=True)
    def _(j):
      sv = sbuf[pl.ds(j * 16, 16)]
      dv = dbuf[pl.ds(j * 16, 16)]
      kbuf[pl.ds(j * 16, 16)] = sv * N + dv
      ebuf[pl.ds(j * 16, 16)] = lax.iota(i32, 16) + (b + j * 16)

    cp = pltpu.async_copy(ebuf, T_out.at[kbuf], sem)
    pltpu.sync_copy(onesb, deg_sh.at[sbuf], add=True)
    cp.wait()

  plsc.subcore_barrier()

  pltpu.sync_copy(deg_sh.at[pl.ds(s * 624, 640)], dsl)
  pltpu.sync_copy(dsl, deg_out.at[pl.ds(c * N + s * 624, 640)])


_k2 = functools.partial(
    pl.kernel,
    out_type=[
        jax.ShapeDtypeStruct((N * N,), i32),
        jax.ShapeDtypeStruct((NC * N,), f32),
    ],
    mesh=_mesh,
    compiler_params=pltpu.CompilerParams(needs_layout_passes=False),
    scratch_types=[
        pltpu.VMEM((CH,), i32),
        pltpu.VMEM((CH,), i32),
        pltpu.VMEM((CH,), i32),
        pltpu.VMEM((CH,), i32),
        pltpu.VMEM((CH,), f32),
        pltpu.VMEM((640,), f32),
        pltpu.VMEM_SHARED((N,), f32),
        pltpu.SemaphoreType.DMA,
    ],
)(_k2_body)


def _dinv_body(deg_ref, dinv_ref):
  d = deg_ref[0] + deg_ref[1]
  dinv_ref[...] = jnp.where(d > 0.0, lax.rsqrt(d), 0.0)


def _dinv(deg):
  r = pl.pallas_call(
      _dinv_body,
      out_shape=jax.ShapeDtypeStruct((80, 125), f32),
  )(deg.reshape(NC, 80, 125))
  return r.reshape(N)


def _k3_body(esrc, edst, T_in, k_out, sbig, dbig, kbig, rbig, kv, rbufs,
             kbufs, onesb, cnt_sh, semg, sems):
  c = lax.axis_index("c")
  s = lax.axis_index("s")
  NB = 10

  @pl.when(c == 0)
  def _():
    @pl.loop(0, CH // 16, unroll=True)
    def _(j):
      onesb[pl.ds(j * 16, 16)] = jnp.ones((16,), f32)

    @plsc.parallel_loop(0, EPT // 16, unroll=10)
    def _(i):
      kv[pl.ds(i * 16, 16)] = jnp.zeros((16,), f32)

    pltpu.sync_copy(kv, cnt_sh.at[pl.ds(s * EPT, EPT)])

    base0 = s * EPT
    pltpu.sync_copy(esrc.at[pl.ds(base0, EPT)], sbig)
    pltpu.sync_copy(edst.at[pl.ds(base0, EPT)], dbig)

    @plsc.parallel_loop(0, EPT // 16, unroll=10)
    def _(i):
      kbig[pl.ds(i * 16, 16)] = sbig[pl.ds(i * 16, 16)] * N + dbig[pl.ds(i * 16, 16)]

  plsc.subcore_barrier()

  NCHUNK = EPT // CH
  NBATCH = NCHUNK // NB

  @pl.when(c == 0)
  def _():
    @pl.loop(0, NBATCH)
    def _(bt):
      b0 = bt * NB * CH

      for j in range(NB):
        @pl.loop(0, CH // 16, unroll=True)
        def _(q):
          kbufs[j, pl.ds(q * 16, 16)] = kbig[pl.ds(b0 + j * CH + q * 16, 16)]
      for j in range(NB):
        pltpu.async_copy(
            T_in.at[kbufs.at[j]],
            rbig.at[pl.ds(b0 + j * CH, CH)], semg)
      for j in range(NB):
        pltpu.make_async_copy(
            T_in.at[kbufs.at[j]],
            rbig.at[pl.ds(b0 + j * CH, CH)], semg).wait()

      for j in range(NB):
        @pl.loop(0, CH // 16, unroll=True)
        def _(q):
          rbufs[j, pl.ds(q * 16, 16)] = rbig[pl.ds(b0 + j * CH + q * 16, 16)]
      for j in range(NB):
        pltpu.async_copy(onesb, cnt_sh.at[rbufs.at[j]], add=True, sem=sems)
      for j in range(NB):
        pltpu.make_async_copy(onesb, cnt_sh.at[rbufs.at[j]], sems).wait()

  plsc.subcore_barrier()

  @pl.when(c == 0)
  def _():
    @pl.loop(0, NBATCH)
    def _(bt):
      b0 = bt * NB * CH
      for j in range(NB):
        @pl.loop(0, CH // 16, unroll=True)
        def _(q):
          rbufs[j, pl.ds(q * 16, 16)] = rbig[pl.ds(b0 + j * CH + q * 16, 16)]
      for j in range(NB):
        pltpu.async_copy(
            cnt_sh.at[rbufs.at[j]],
            kv.at[pl.ds(b0 + j * CH, CH)], semg)
      for j in range(NB):
        pltpu.make_async_copy(
            cnt_sh.at[rbufs.at[j]],
            kv.at[pl.ds(b0 + j * CH, CH)], semg).wait()

    pltpu.sync_copy(kv, k_out.at[pl.ds(s * EPT, EPT)])


_k3 = functools.partial(
    pl.kernel,
    out_type=jax.ShapeDtypeStruct((E,), f32),
    mesh=_mesh,
    compiler_params=pltpu.CompilerParams(needs_layout_passes=False),
    scratch_types=[
        pltpu.VMEM((EPT,), i32),
        pltpu.VMEM((EPT,), i32),
        pltpu.VMEM((EPT,), i32),
        pltpu.VMEM((EPT,), i32),
        pltpu.VMEM((EPT,), f32),
        pltpu.VMEM((10, CH), i32),
        pltpu.VMEM((10, CH), i32),
        pltpu.VMEM((CH,), f32),
        pltpu.VMEM_SHARED((E,), f32),
        pltpu.SemaphoreType.DMA,
        pltpu.SemaphoreType.DMA,
    ],
)(_k3_body)


def _k4_body(esrc, edst, karr, dinv, P1, T2, sheaf_out, opart_out, sbuf, dbuf,
             kcb, coefb, dglb, stg, dsb, ddb, a1, b2, shv, dinv_sh, acc_sh,
             sem1, sem2):
  c = lax.axis_index("c")
  s = lax.axis_index("s")
  w = c * NS + s

  pltpu.sync_copy(dinv.at[pl.ds(s * 624, 640)], stg)
  pltpu.sync_copy(stg, dinv_sh.at[pl.ds(s * 624, 640)])
  for h in range(8):
    @pl.loop(0, CH, unroll=8)
    def _(i):
      shv[i, pl.ds(h * 16, 16)] = jnp.zeros((16,), f32)
  for q in range(8):
    pltpu.sync_copy(shv, acc_sh.at[pl.ds(s * 624 + q * CH, CH)])

  plsc.subcore_barrier()

  base0 = w * EPW

  @pl.loop(0, EPW // CH)
  def _(g):
    b = base0 + g * CH
    pltpu.sync_copy(esrc.at[pl.ds(b, CH)], sbuf)
    pltpu.sync_copy(edst.at[pl.ds(b, CH)], dbuf)
    pltpu.sync_copy(karr.at[pl.ds(b, CH)], kcb)
    cp1 = pltpu.async_copy(P1.at[sbuf], a1, sem1)
    cp2 = pltpu.async_copy(T2.at[dbuf], b2, sem2)
    pltpu.sync_copy(dinv_sh.at[sbuf], dsb)
    pltpu.sync_copy(dinv_sh.at[dbuf], ddb)

    @plsc.parallel_loop(0, CH // 16, unroll=5)
    def _(j):
      sv = sbuf[pl.ds(j * 16, 16)]
      dv = dbuf[pl.ds(j * 16, 16)]
      lv = -(dsb[pl.ds(j * 16, 16)] * ddb[pl.ds(j * 16, 16)])
      diag = jnp.where(sv == dv, 1.0, 0.0).astype(f32)
      coefb[pl.ds(j * 16, 16)] = -(kcb[pl.ds(j * 16, 16)] * lv + diag)
      dglb[pl.ds(j * 16, 16)] = -(diag * lv)

    cp1.wait()
    cp2.wait()

    @plsc.parallel_loop(0, CH, unroll=4)
    def _(i):
      for h in range(8):
        q = a1[i, pl.ds(h * 16, 16)] + b2[i, pl.ds(h * 16, 16)]
        ee = jnp.exp(q + q)
        shv[i, pl.ds(h * 16, 16)] = 1.0 - 2.0 / (ee + 1.0)

    pltpu.sync_copy(shv, sheaf_out.at[pl.ds(b, CH)])

    @plsc.parallel_loop(0, CH, unroll=4)
    def _(i):
      bi = jnp.full((16,), 0, i32) + i
      cf = plsc.load_gather(coefb, [bi])
      dg = plsc.load_gather(dglb, [bi])
      for h in range(8):
        sw = shv[i, pl.ds(h * 16, 16)]
        vd = b2[i, pl.ds(COUT + h * 16, 16)]
        shv[i, pl.ds(h * 16, 16)] = (cf * sw + dg) * vd

    pltpu.sync_copy(shv, acc_sh.at[sbuf], add=True)

  plsc.subcore_barrier()
  pltpu.sync_copy(
      acc_sh.at[pl.ds(s * 624, 640)], opart_out.at[c, pl.ds(s * 624, 640)])


_k4 = functools.partial(
    pl.kernel,
    out_type=[
        jax.ShapeDtypeStruct((E, COUT), f32),
        jax.ShapeDtypeStruct((NC, N, COUT), f32),
    ],
    mesh=_mesh,
    compiler_params=pltpu.CompilerParams(needs_layout_passes=False),
    scratch_types=[
        pltpu.VMEM((CH,), i32),
        pltpu.VMEM((CH,), i32),
        pltpu.VMEM((CH,), f32),
        pltpu.VMEM((CH,), f32),
        pltpu.VMEM((CH,), f32),
        pltpu.VMEM((640,), f32),
        pltpu.VMEM((CH,), f32),
        pltpu.VMEM((CH,), f32),
        pltpu.VMEM((CH, COUT), f32),
        pltpu.VMEM((CH, 2 * COUT), f32),
        pltpu.VMEM((CH, COUT), f32),
        pltpu.VMEM_SHARED((N,), f32),
        pltpu.VMEM_SHARED((N, COUT), f32),
        pltpu.SemaphoreType.DMA,
        pltpu.SemaphoreType.DMA,
    ],
)(_k4_body)


def _sum_body(a_ref, b_ref, o_ref):
  o_ref[...] = a_ref[...] + b_ref[...]


def _sum(a, b):
  return pl.pallas_call(
      _sum_body,
      grid=(N // _BLK,),
      in_specs=[
          pl.BlockSpec((_BLK, COUT), lambda i: (i, 0)),
          pl.BlockSpec((_BLK, COUT), lambda i: (i, 0)),
      ],
      out_specs=pl.BlockSpec((_BLK, COUT), lambda i: (i, 0)),
      out_shape=jax.ShapeDtypeStruct((N, COUT), f32),
  )(a, b)


@jax.jit
def _impl(x, edge_index, W_edge, b_edge, W_lin):
  be2d = b_edge.reshape(1, COUT)
  P1, T2 = _prep(x, W_edge, be2d, W_lin)
  esrc = edge_index[0]
  edst = edge_index[1]
  T, deg = _k2(esrc, edst)
  dinv = _dinv(deg)
  karr = _k3(esrc, edst, T)
  sheaf, opart = _k4(esrc, edst, karr, dinv, P1, T2)
  out = _sum(opart[0], opart[1])
  return out, sheaf


def kernel(x, edge_index, W_edge, b_edge, W_lin):
  return _impl(x, edge_index, W_edge, b_edge, W_lin)


# speedup vs baseline: 1.0923x; 1.0923x over previous
import functools

import jax
import jax.numpy as jnp
from jax import lax
from jax.experimental import pallas as pl
from jax.experimental.pallas import tpu as pltpu
from jax.experimental.pallas import tpu_sc as plsc

N = 10000
E = 320000
CIN = 128
COUT = 128
NC = 2
NS = 16
EPW = E // (NC * NS)
EPT = E // NS
CH = 80
NPT = N // NS

f32 = jnp.float32
i32 = jnp.int32

_mesh = plsc.VectorSubcoreMesh(
    core_axis_name="c", subcore_axis_name="s", num_cores=NC, num_subcores=NS)


_BLK = 2000


def _prep_body(x_ref, we_ref, be_ref, wl_ref, p1_ref, t2_ref):
  xb = x_ref[...]
  dn = (((1,), (1,)), ((), ()))
  w1 = we_ref[:, :CIN]
  w2 = we_ref[:, CIN:]
  p1_ref[...] = lax.dot_general(xb, w1, dn, preferred_element_type=f32)
  t2_ref[:, :COUT] = (
      lax.dot_general(xb, w2, dn, preferred_element_type=f32) + be_ref[...])
  t2_ref[:, COUT:] = lax.dot_general(
      xb, wl_ref[...], dn, preferred_element_type=f32)


def _prep(x, W_edge, be2d, W_lin):
  return pl.pallas_call(
      _prep_body,
      grid=(N // _BLK,),
      in_specs=[
          pl.BlockSpec((_BLK, CIN), lambda i: (i, 0)),
          pl.BlockSpec((COUT, 2 * CIN), lambda i: (0, 0)),
          pl.BlockSpec((1, COUT), lambda i: (0, 0)),
          pl.BlockSpec((COUT, CIN), lambda i: (0, 0)),
      ],
      out_specs=[
          pl.BlockSpec((_BLK, COUT), lambda i: (i, 0)),
          pl.BlockSpec((_BLK, 2 * COUT), lambda i: (i, 0)),
      ],
      out_shape=[
          jax.ShapeDtypeStruct((N, COUT), f32),
          jax.ShapeDtypeStruct((N, 2 * COUT), f32),
      ],
  )(x, W_edge, be2d, W_lin)


def _k2_body(esrc, edst, T_out, deg_out, sbig, dbig, kbig, ebig, kbufs, sbufs,
             onesb, dsl, deg_sh, semg, sems):
  c = lax.axis_index("c")
  s = lax.axis_index("s")
  w = c * NS + s
  NB = 5

  @pl.loop(0, 40, unroll=True)
  def _(i):
    dsl[pl.ds(i * 16, 16)] = jnp.zeros((16,), f32)
  pltpu.sync_copy(dsl, deg_sh.at[pl.ds(s * 624, 640)])

  @pl.loop(0, CH // 16, unroll=True)
  def _(j):
    onesb[pl.ds(j * 16, 16)] = jnp.ones((16,), f32)

  base0 = w * EPW
  pltpu.sync_copy(esrc.at[pl.ds(base0, EPW)], sbig)
  pltpu.sync_copy(edst.at[pl.ds(base0, EPW)], dbig)

  @plsc.parallel_loop(0, EPW // 16, unroll=5)
  def _(i):
    kbig[pl.ds(i * 16, 16)] = sbig[pl.ds(i * 16, 16)] * N + dbig[pl.ds(i * 16, 16)]
    ebig[pl.ds(i * 16, 16)] = lax.iota(i32, 16) + (base0 + i * 16)

  plsc.subcore_barrier()

  @pl.loop(0, (EPW // CH) // NB)
  def _(bt):
    b0 = bt * NB * CH
    for j in range(NB):
      @pl.loop(0, CH // 16, unroll=True)
      def _(q):
        kbufs[j, pl.ds(q * 16, 16)] = kbig[pl.ds(b0 + j * CH + q * 16, 16)]
        sbufs[j, pl.ds(q * 16, 16)] = sbig[pl.ds(b0 + j * CH + q * 16, 16)]
    for j in range(NB):
      pltpu.async_copy(
          ebig.at[pl.ds(b0 + j * CH, CH)], T_out.at[kbufs.at[j]], semg)
      pltpu.async_copy(onesb, deg_sh.at[sbufs.at[j]], add=True, sem=sems)
    for j in range(NB):
      pltpu.make_async_copy(
          ebig.at[pl.ds(b0 + j * CH, CH)], T_out.at[kbufs.at[j]], semg).wait()
      pltpu.make_async_copy(onesb, deg_sh.at[sbufs.at[j]], sems).wait()

  plsc.subcore_barrier()

  pltpu.sync_copy(deg_sh.at[pl.ds(s * 624, 640)], dsl)
  pltpu.sync_copy(dsl, deg_out.at[pl.ds(c * N + s * 624, 640)])


_k2 = functools.partial(
    pl.kernel,
    out_type=[
        jax.ShapeDtypeStruct((N * N,), i32),
        jax.ShapeDtypeStruct((NC * N,), f32),
    ],
    mesh=_mesh,
    compiler_params=pltpu.CompilerParams(needs_layout_passes=False),
    scratch_types=[
        pltpu.VMEM((EPW,), i32),
        pltpu.VMEM((EPW,), i32),
        pltpu.VMEM((EPW,), i32),
        pltpu.VMEM((EPW,), i32),
        pltpu.VMEM((5, CH), i32),
        pltpu.VMEM((5, CH), i32),
        pltpu.VMEM((CH,), f32),
        pltpu.VMEM((640,), f32),
        pltpu.VMEM_SHARED((N,), f32),
        pltpu.SemaphoreType.DMA,
        pltpu.SemaphoreType.DMA,
    ],
)(_k2_body)


def _dinv_body(deg_ref, dinv_ref):
  d = deg_ref[0] + deg_ref[1]
  dinv_ref[...] = jnp.where(d > 0.0, lax.rsqrt(d), 0.0)


def _dinv(deg):
  r = pl.pallas_call(
      _dinv_body,
      out_shape=jax.ShapeDtypeStruct((80, 125), f32),
  )(deg.reshape(NC, 80, 125))
  return r.reshape(N)


def _k3_body(esrc, edst, T_in, k_out, sbig, dbig, kbig, rbig, kv, rbufs,
             kbufs, onesb, cnt_sh, semg, sems):
  c = lax.axis_index("c")
  s = lax.axis_index("s")
  NB = 10

  @pl.when(c == 0)
  def _():
    @pl.loop(0, CH // 16, unroll=True)
    def _(j):
      onesb[pl.ds(j * 16, 16)] = jnp.ones((16,), f32)

    @plsc.parallel_loop(0, EPT // 16, unroll=10)
    def _(i):
      kv[pl.ds(i * 16, 16)] = jnp.zeros((16,), f32)

    pltpu.sync_copy(kv, cnt_sh.at[pl.ds(s * EPT, EPT)])

    base0 = s * EPT
    pltpu.sync_copy(esrc.at[pl.ds(base0, EPT)], sbig)
    pltpu.sync_copy(edst.at[pl.ds(base0, EPT)], dbig)

    @plsc.parallel_loop(0, EPT // 16, unroll=10)
    def _(i):
      kbig[pl.ds(i * 16, 16)] = sbig[pl.ds(i * 16, 16)] * N + dbig[pl.ds(i * 16, 16)]

  plsc.subcore_barrier()

  NCHUNK = EPT // CH
  NBATCH = NCHUNK // NB

  @pl.when(c == 0)
  def _():
    @pl.loop(0, NBATCH)
    def _(bt):
      b0 = bt * NB * CH

      for j in range(NB):
        @pl.loop(0, CH // 16, unroll=True)
        def _(q):
          kbufs[j, pl.ds(q * 16, 16)] = kbig[pl.ds(b0 + j * CH + q * 16, 16)]
      for j in range(NB):
        pltpu.async_copy(
            T_in.at[kbufs.at[j]],
            rbig.at[pl.ds(b0 + j * CH, CH)], semg)
      for j in range(NB):
        pltpu.make_async_copy(
            T_in.at[kbufs.at[j]],
            rbig.at[pl.ds(b0 + j * CH, CH)], semg).wait()

      for j in range(NB):
        @pl.loop(0, CH // 16, unroll=True)
        def _(q):
          rbufs[j, pl.ds(q * 16, 16)] = rbig[pl.ds(b0 + j * CH + q * 16, 16)]
      for j in range(NB):
        pltpu.async_copy(onesb, cnt_sh.at[rbufs.at[j]], add=True, sem=sems)
      for j in range(NB):
        pltpu.make_async_copy(onesb, cnt_sh.at[rbufs.at[j]], sems).wait()

  plsc.subcore_barrier()

  @pl.when(c == 0)
  def _():
    @pl.loop(0, NBATCH)
    def _(bt):
      b0 = bt * NB * CH
      for j in range(NB):
        @pl.loop(0, CH // 16, unroll=True)
        def _(q):
          rbufs[j, pl.ds(q * 16, 16)] = rbig[pl.ds(b0 + j * CH + q * 16, 16)]
      for j in range(NB):
        pltpu.async_copy(
            cnt_sh.at[rbufs.at[j]],
            kv.at[pl.ds(b0 + j * CH, CH)], semg)
      for j in range(NB):
        pltpu.make_async_copy(
            cnt_sh.at[rbufs.at[j]],
            kv.at[pl.ds(b0 + j * CH, CH)], semg).wait()

    pltpu.sync_copy(kv, k_out.at[pl.ds(s * EPT, EPT)])


_k3 = functools.partial(
    pl.kernel,
    out_type=jax.ShapeDtypeStruct((E,), f32),
    mesh=_mesh,
    compiler_params=pltpu.CompilerParams(needs_layout_passes=False),
    scratch_types=[
        pltpu.VMEM((EPT,), i32),
        pltpu.VMEM((EPT,), i32),
        pltpu.VMEM((EPT,), i32),
        pltpu.VMEM((EPT,), i32),
        pltpu.VMEM((EPT,), f32),
        pltpu.VMEM((10, CH), i32),
        pltpu.VMEM((10, CH), i32),
        pltpu.VMEM((CH,), f32),
        pltpu.VMEM_SHARED((E,), f32),
        pltpu.SemaphoreType.DMA,
        pltpu.SemaphoreType.DMA,
    ],
)(_k3_body)


def _k4_body(esrc, edst, karr, dinv, P1, T2, sheaf_out, opart_out, sb4, db4,
             kc4, dvs, dvd, sidx, coefb, dglb, stg, a1, b2, shv, dinv_sh,
             acc_sh, semr, semw, sema):
  c = lax.axis_index("c")
  s = lax.axis_index("s")
  w = c * NS + s
  SCHN = 5 * CH

  pltpu.sync_copy(dinv.at[pl.ds(s * 624, 640)], stg)
  pltpu.sync_copy(stg, dinv_sh.at[pl.ds(s * 624, 640)])
  for h in range(8):
    @plsc.parallel_loop(0, CH, unroll=8)
    def _(i):
      shv[i, pl.ds(h * 16, 16)] = jnp.zeros((16,), f32)
  for q in range(8):
    pltpu.sync_copy(shv, acc_sh.at[pl.ds(s * 624 + q * CH, CH)])

  plsc.subcore_barrier()

  base0 = w * EPW

  @pl.loop(0, EPW // SCHN)
  def _(g):
    bs = base0 + g * SCHN
    pltpu.sync_copy(esrc.at[pl.ds(bs, SCHN)], sb4)
    pltpu.sync_copy(edst.at[pl.ds(bs, SCHN)], db4)
    pltpu.sync_copy(karr.at[pl.ds(bs, SCHN)], kc4)
    for j in range(5):
      pltpu.async_copy(
          dinv_sh.at[sb4.at[pl.ds(j * CH, CH)]],
          dvs.at[pl.ds(j * CH, CH)], sema)
      pltpu.async_copy(
          dinv_sh.at[db4.at[pl.ds(j * CH, CH)]],
          dvd.at[pl.ds(j * CH, CH)], sema)
    for j in range(5):
      pltpu.make_async_copy(
          dinv_sh.at[sb4.at[pl.ds(j * CH, CH)]],
          dvs.at[pl.ds(j * CH, CH)], sema).wait()
      pltpu.make_async_copy(
          dinv_sh.at[db4.at[pl.ds(j * CH, CH)]],
          dvd.at[pl.ds(j * CH, CH)], sema).wait()
    for j in range(5):
      @pl.loop(0, CH // 16, unroll=True)
      def _(q):
        sidx[j, pl.ds(q * 16, 16)] = sb4[pl.ds(j * CH + q * 16, 16)]

    for j in range(5):
      b = bs + j * CH
      if j > 0:
        pltpu.make_async_copy(
            shv, sheaf_out.at[pl.ds(b - CH, CH)], semw).wait()
        pltpu.make_async_copy(a1, acc_sh.at[sidx.at[j - 1]], semw).wait()
      else:
        @pl.when(g > 0)
        def _():
          pltpu.make_async_copy(
              shv, sheaf_out.at[pl.ds(b - CH, CH)], semw).wait()
          pltpu.make_async_copy(a1, acc_sh.at[sidx.at[4]], semw).wait()

      cp1 = pltpu.async_copy(P1.at[sb4.at[pl.ds(j * CH, CH)]], a1, semr)
      cp2 = pltpu.async_copy(T2.at[db4.at[pl.ds(j * CH, CH)]], b2, semr)

      @plsc.parallel_loop(0, CH // 16, unroll=5)
      def _(q):
        sv = sb4[pl.ds(j * CH + q * 16, 16)]
        dv = db4[pl.ds(j * CH + q * 16, 16)]
        lv = -(dvs[pl.ds(j * CH + q * 16, 16)] * dvd[pl.ds(j * CH + q * 16, 16)])
        diag = jnp.where(sv == dv, 1.0, 0.0).astype(f32)
        coefb[pl.ds(q * 16, 16)] = -(kc4[pl.ds(j * CH + q * 16, 16)] * lv + diag)
        dglb[pl.ds(q * 16, 16)] = -(diag * lv)

      cp1.wait()
      cp2.wait()

      @plsc.parallel_loop(0, CH, unroll=4)
      def _(i):
        for h in range(8):
          qq = a1[i, pl.ds(h * 16, 16)] + b2[i, pl.ds(h * 16, 16)]
          ee = jnp.exp(qq + qq)
          shv[i, pl.ds(h * 16, 16)] = 1.0 - 2.0 / (ee + 1.0)

      pltpu.async_copy(shv, sheaf_out.at[pl.ds(b, CH)], semw)

      @plsc.parallel_loop(0, CH, unroll=4)
      def _(i):
        bi = jnp.full((16,), 0, i32) + i
        cf = plsc.load_gather(coefb, [bi])
        dg = plsc.load_gather(dglb, [bi])
        for h in range(8):
          sw = shv[i, pl.ds(h * 16, 16)]
          vd = b2[i, pl.ds(COUT + h * 16, 16)]
          a1[i, pl.ds(h * 16, 16)] = (cf * sw + dg) * vd

      pltpu.async_copy(a1, acc_sh.at[sidx.at[j]], add=True, sem=semw)

  pltpu.make_async_copy(
      shv, sheaf_out.at[pl.ds(base0 + EPW - CH, CH)], semw).wait()
  pltpu.make_async_copy(a1, acc_sh.at[sidx.at[4]], semw).wait()

  plsc.subcore_barrier()
  pltpu.sync_copy(
      acc_sh.at[pl.ds(s * 624, 640)], opart_out.at[c, pl.ds(s * 624, 640)])


_k4 = functools.partial(
    pl.kernel,
    out_type=[
        jax.ShapeDtypeStruct((E, COUT), f32),
        jax.ShapeDtypeStruct((NC, N, COUT), f32),
    ],
    mesh=_mesh,
    compiler_params=pltpu.CompilerParams(needs_layout_passes=False),
    scratch_types=[
        pltpu.VMEM((400,), i32),
        pltpu.VMEM((400,), i32),
        pltpu.VMEM((400,), f32),
        pltpu.VMEM((400,), f32),
        pltpu.VMEM((400,), f32),
        pltpu.VMEM((5, CH), i32),
        pltpu.VMEM((CH,), f32),
        pltpu.VMEM((CH,), f32),
        pltpu.VMEM((640,), f32),
        pltpu.VMEM((CH, COUT), f32),
        pltpu.VMEM((CH, 2 * COUT), f32),
        pltpu.VMEM((CH, COUT), f32),
        pltpu.VMEM_SHARED((N,), f32),
        pltpu.VMEM_SHARED((N, COUT), f32),
        pltpu.SemaphoreType.DMA,
        pltpu.SemaphoreType.DMA,
        pltpu.SemaphoreType.DMA,
    ],
)(_k4_body)


def _sum_body(a_ref, b_ref, o_ref):
  o_ref[...] = a_ref[...] + b_ref[...]


def _sum(a, b):
  return pl.pallas_call(
      _sum_body,
      grid=(N // _BLK,),
      in_specs=[
          pl.BlockSpec((_BLK, COUT), lambda i: (i, 0)),
          pl.BlockSpec((_BLK, COUT), lambda i: (i, 0)),
      ],
      out_specs=pl.BlockSpec((_BLK, COUT), lambda i: (i, 0)),
      out_shape=jax.ShapeDtypeStruct((N, COUT), f32),
  )(a, b)


@jax.jit
def _impl(x, edge_index, W_edge, b_edge, W_lin):
  be2d = b_edge.reshape(1, COUT)
  P1, T2 = _prep(x, W_edge, be2d, W_lin)
  esrc = edge_index[0]
  edst = edge_index[1]
  T, deg = _k2(esrc, edst)
  dinv = _dinv(deg)
  karr = _k3(esrc, edst, T)
  sheaf, opart = _k4(esrc, edst, karr, dinv, P1, T2)
  out = _sum(opart[0], opart[1])
  return out, sheaf


def kernel(x, edge_index, W_edge, b_edge, W_lin):
  return _impl(x, edge_index, W_edge, b_edge, W_lin)
